# Initial kernel scaffold; baseline (speedup 1.0000x reference)
import jax
import jax.numpy as jnp
from jax.experimental import pallas as pl


def kernel(scores, k):
    raise NotImplementedError("write your pallas kernel here")


# speedup vs baseline: 1.6256x; 1.6256x over previous
import functools

import jax
import jax.numpy as jnp
from jax import lax
from jax.experimental import pallas as pl
from jax.experimental.pallas import tpu as pltpu
from jax.experimental.pallas import tpu_sc as plsc

R, N, L = 32, 32768, 16
NB = 256
NC, NS = 2, 16
MINT32 = -2**31


def _sortable_key(v):
    b = plsc.bitcast(v, jnp.int32)
    m = (b >> 31) | jnp.int32(MINT32)
    return b ^ m


def _select_bin(total_ref, k_rem):
    def body(c, carry):
        cum, found, t, knext = carry
        cd = 15 - c
        v = total_ref[pl.ds(cd * 16, 16)]
        sfx = lax.rev(plsc.cumsum(lax.rev(v, (0,))), (0,))
        abs_sfx = sfx + cum
        m = abs_sfx >= k_rem
        cnt = jnp.sum(m.astype(jnp.int32))
        has = cnt > 0
        t_loc = cnt - 1
        onehot = lax.iota(jnp.int32, 16) == t_loc
        tot_t = jnp.sum(jnp.where(onehot, v, 0))
        sfxa_t = jnp.sum(jnp.where(onehot, abs_sfx, 0))
        take = jnp.logical_and(found == 0, has)
        t = jnp.where(take, cd * 16 + t_loc, t)
        knext = jnp.where(take, k_rem - (sfxa_t - tot_t), knext)
        found = jnp.where(has, jnp.int32(1), found)
        cum = cum + jnp.sum(v)
        return cum, found, t, knext
    z = jnp.int32(0)
    _, _, t, knext = lax.fori_loop(0, 16, body, (z, z, z, z))
    return t, knext


def _sc_body(scores_hbm, kvec_hbm, out_hbm, row_v, hist_v, total_v, kv_v):
    wid = lax.axis_index("s") * NC + lax.axis_index("c")
    pltpu.sync_copy(scores_hbm.at[wid], row_v)
    pltpu.sync_copy(kvec_hbm, kv_v)
    k_rem = kv_v[...][0]

    lane = lax.iota(jnp.int32, L)
    ones = jnp.ones((L,), jnp.int32)
    prefix = jnp.int32(0)

    for level in range(4):
        shift = 24 - 8 * level

        def zero_body(i, _):
            hist_v[pl.ds(i * 16, 16)] = jnp.zeros((16,), jnp.int32)
            return 0
        lax.fori_loop(0, (L * NB) // 16, zero_body, 0)

        def hist_body(i, _, shift=shift, prefix=prefix, level=level):
            key = _sortable_key(row_v[pl.ds(i * L, L)])
            bins = lax.shift_right_logical(key, shift) & 0xFF
            idx = lane * NB + bins
            if level == 0:
                plsc.addupdate_scatter(hist_v, [idx], ones)
            else:
                match = lax.shift_right_logical(key, shift + 8) == prefix
                plsc.addupdate_scatter(hist_v, [idx], ones, mask=match)
            return 0
        lax.fori_loop(0, N // L, hist_body, 0)

        def merge_body(c, _):
            def lane_body(l, acc):
                return acc + hist_v[pl.ds(l * NB + c * 16, 16)]
            total_v[pl.ds(c * 16, 16)] = lax.fori_loop(
                0, L, lane_body, jnp.zeros((16,), jnp.int32))
            return 0
        lax.fori_loop(0, NB // 16, merge_body, 0)

        t, k_rem = _select_bin(total_v, k_rem)
        prefix = lax.shift_left(prefix, 8) | t

    bmask = jnp.where(prefix < 0, jnp.int32(MINT32), jnp.int32(-1))
    tbits = jnp.broadcast_to(prefix ^ bmask, (L,))
    thresh = plsc.bitcast(tbits, jnp.float32)

    def mask_body(i, _):
        v = row_v[pl.ds(i * L, L)]
        row_v[pl.ds(i * L, L)] = jnp.where(v >= thresh, v, jnp.float32(-1e9))
        return 0
    lax.fori_loop(0, N // L, mask_body, 0)

    pltpu.sync_copy(row_v, out_hbm.at[wid])


_sc_topk_mask = functools.partial(
    pl.kernel,
    out_type=jax.ShapeDtypeStruct((R, N), jnp.float32),
    mesh=plsc.VectorSubcoreMesh(
        core_axis_name="c", subcore_axis_name="s",
        num_cores=NC, num_subcores=NS),
    compiler_params=pltpu.CompilerParams(needs_layout_passes=False),
    scratch_types=[
        pltpu.VMEM((N,), jnp.float32),
        pltpu.VMEM((L * NB,), jnp.int32),
        pltpu.VMEM((NB,), jnp.int32),
        pltpu.VMEM((L,), jnp.int32),
    ],
)(_sc_body)


def kernel(scores, k):
    kvec = jnp.full((L,), k, jnp.int32)
    return _sc_topk_mask(scores, kvec)


# speedup vs baseline: 1.9003x; 1.1690x over previous
import functools

import jax
import jax.numpy as jnp
from jax import lax
from jax.experimental import pallas as pl
from jax.experimental.pallas import tpu as pltpu
from jax.experimental.pallas import tpu_sc as plsc

R, N, L = 32, 32768, 16
NB = 256
NC, NS = 2, 16
MINT32 = -2**31


def _sortable_key(v):
    b = plsc.bitcast(v, jnp.int32)
    m = (b >> 31) | jnp.int32(MINT32)
    return b ^ m


def _select_bin(total_ref, k_rem):
    def body(c, carry):
        cum, found, t, knext = carry
        cd = 15 - c
        v = total_ref[pl.ds(cd * 16, 16)]
        sfx = lax.rev(plsc.cumsum(lax.rev(v, (0,))), (0,))
        abs_sfx = sfx + cum
        m = abs_sfx >= k_rem
        cnt = jnp.sum(m.astype(jnp.int32))
        has = cnt > 0
        t_loc = cnt - 1
        onehot = lax.iota(jnp.int32, 16) == t_loc
        tot_t = jnp.sum(jnp.where(onehot, v, 0))
        sfxa_t = jnp.sum(jnp.where(onehot, abs_sfx, 0))
        take = jnp.logical_and(found == 0, has)
        t = jnp.where(take, cd * 16 + t_loc, t)
        knext = jnp.where(take, k_rem - (sfxa_t - tot_t), knext)
        found = jnp.where(has, jnp.int32(1), found)
        cum = cum + jnp.sum(v)
        return cum, found, t, knext
    z = jnp.int32(0)
    _, _, t, knext = lax.fori_loop(0, 16, body, (z, z, z, z))
    return t, knext


def _sc_body(scores_hbm, kvec_hbm, out_hbm, row_v, hist_v, total_v, kv_v):
    wid = lax.axis_index("s") * NC + lax.axis_index("c")
    pltpu.sync_copy(scores_hbm.at[wid], row_v)
    pltpu.sync_copy(kvec_hbm, kv_v)
    k_rem = kv_v[...][0]

    lane = lax.iota(jnp.int32, L)
    ones = jnp.ones((L,), jnp.int32)
    prefix = jnp.int32(0)

    U = 8
    zeros16 = jnp.zeros((16,), jnp.int32)

    for level in range(4):
        shift = 24 - 8 * level

        def zero_body(i, _):
            for u in range(U):
                hist_v[pl.ds((i * U + u) * 16, 16)] = zeros16
            return 0
        lax.fori_loop(0, (L * NB) // (16 * U), zero_body, 0)

        def hist_body(i, _, shift=shift, prefix=prefix, level=level):
            for u in range(U):
                key = _sortable_key(row_v[pl.ds((i * U + u) * L, L)])
                bins = lax.shift_right_logical(key, shift) & 0xFF
                idx = lane * NB + bins
                if level == 0:
                    plsc.addupdate_scatter(hist_v, [idx], ones)
                else:
                    match = lax.shift_right_logical(key, shift + 8) == prefix
                    plsc.addupdate_scatter(hist_v, [idx], ones, mask=match)
            return 0
        lax.fori_loop(0, N // (L * U), hist_body, 0)

        def merge_body(c, _):
            acc = zeros16
            for l in range(L):
                acc = acc + hist_v[pl.ds(l * NB + c * 16, 16)]
            total_v[pl.ds(c * 16, 16)] = acc
            return 0
        lax.fori_loop(0, NB // 16, merge_body, 0)

        t, k_rem = _select_bin(total_v, k_rem)
        prefix = lax.shift_left(prefix, 8) | t

    bmask = jnp.where(prefix < 0, jnp.int32(MINT32), jnp.int32(-1))
    tbits = jnp.broadcast_to(prefix ^ bmask, (L,))
    thresh = plsc.bitcast(tbits, jnp.float32)

    def mask_body(i, _):
        for u in range(U):
            v = row_v[pl.ds((i * U + u) * L, L)]
            row_v[pl.ds((i * U + u) * L, L)] = jnp.where(
                v >= thresh, v, jnp.float32(-1e9))
        return 0
    lax.fori_loop(0, N // (L * U), mask_body, 0)

    pltpu.sync_copy(row_v, out_hbm.at[wid])


_sc_topk_mask = functools.partial(
    pl.kernel,
    out_type=jax.ShapeDtypeStruct((R, N), jnp.float32),
    mesh=plsc.VectorSubcoreMesh(
        core_axis_name="c", subcore_axis_name="s",
        num_cores=NC, num_subcores=NS),
    compiler_params=pltpu.CompilerParams(needs_layout_passes=False),
    scratch_types=[
        pltpu.VMEM((N,), jnp.float32),
        pltpu.VMEM((L * NB,), jnp.int32),
        pltpu.VMEM((NB,), jnp.int32),
        pltpu.VMEM((L,), jnp.int32),
    ],
)(_sc_body)


def kernel(scores, k):
    kvec = jnp.full((L,), k, jnp.int32)
    return _sc_topk_mask(scores, kvec)


# speedup vs baseline: 3.0360x; 1.5976x over previous
import functools

import jax
import jax.numpy as jnp
from jax import lax
from jax.experimental import pallas as pl
from jax.experimental.pallas import tpu as pltpu
from jax.experimental.pallas import tpu_sc as plsc

R, N, L = 32, 32768, 16
NB = 256
NBP = NB + 1
NC, NS = 2, 16
MINT32 = -2**31
HIST_WORDS = 4224


def _sortable_key(v):
    b = plsc.bitcast(v, jnp.int32)
    m = (b >> 31) | jnp.int32(MINT32)
    return b ^ m


def _suffix_pick(v, k):
    sfx = lax.rev(plsc.cumsum(lax.rev(v, (0,))), (0,))
    m = sfx >= k
    cnt = plsc.all_reduce_population_count(m)[0]
    idx = cnt - 1
    onehot = lax.iota(jnp.int32, 16) == idx
    val = jnp.sum(jnp.where(onehot, v, 0))
    sfx_i = jnp.sum(jnp.where(onehot, sfx, 0))
    kp = k - (sfx_i - val)
    return idx, kp, val


def _sc_body(scores_hbm, kvec_hbm, out_hbm, row_v, hist_v, total_v, kv_v,
             c1_v, c2_v):
    wid = lax.axis_index("s") * NC + lax.axis_index("c")
    pltpu.sync_copy(scores_hbm.at[wid], row_v)
    pltpu.sync_copy(kvec_hbm, kv_v)
    k_rem = kv_v[...][0]

    lane = lax.iota(jnp.int32, L)
    ones = jnp.ones((L,), jnp.int32)
    zeros16 = jnp.zeros((16,), jnp.int32)
    U = 8

    def zero_hist():
        def zero_body(i, _):
            for u in range(U):
                hist_v[pl.ds((i * U + u) * 16, 16)] = zeros16
            return 0
        lax.fori_loop(0, HIST_WORDS // (16 * U), zero_body, 0)

    def merge_hist():
        def merge_body(c, chunks):
            acc = zeros16
            for l in range(L):
                acc = acc + hist_v[pl.ds(l * NBP + c * 16, 16)]
            total_v[pl.ds(c * 16, 16)] = acc
            return jnp.where(lane == c, jnp.sum(acc), chunks)
        return lax.fori_loop(0, NB // 16, merge_body, zeros16)

    def select(chunks, k):
        cstar, kp, _ = _suffix_pick(chunks, k)
        v = total_v[pl.ds(cstar * 16, 16)]
        t_loc, knext, _ = _suffix_pick(v, kp)
        return cstar * 16 + t_loc, knext

    zero_hist()

    def hist0_body(i, _):
        for u in range(U):
            key = _sortable_key(row_v[pl.ds((i * U + u) * L, L)])
            bins = lax.shift_right_logical(key, 24)
            plsc.addupdate_scatter(hist_v, [lane * NBP + bins], ones)
        return 0
    lax.fori_loop(0, N // (L * U), hist0_body, 0)

    t, k_rem = select(merge_hist(), k_rem)
    prefix = t

    def compact0_body(i, off):
        for u in range(U):
            key = _sortable_key(row_v[pl.ds((i * U + u) * L, L)])
            match = lax.shift_right_logical(key, 24) == prefix
            plsc.store_compressed(c1_v.at[pl.ds(off, L)], key, mask=match)
            off = off + plsc.all_reduce_population_count(match)[0]
        return off
    m_cand = lax.fori_loop(0, N // (L * U), compact0_body, jnp.int32(0))

    bufs = (c1_v, c2_v)
    for level in range(1, 4):
        shift = 24 - 8 * level
        src, dst = bufs[(level - 1) % 2], bufs[level % 2]
        nblk = (m_cand + (L - 1)) // L
        zero_hist()

        def histl_body(i, _, src=src, shift=shift, m_cand=m_cand):
            key = src[pl.ds(i * L, L)]
            valid = (i * L + lane) < m_cand
            bins = lax.shift_right_logical(key, shift) & 0xFF
            plsc.addupdate_scatter(hist_v, [lane * NBP + bins], ones,
                                   mask=valid)
            return 0
        lax.fori_loop(0, nblk, histl_body, 0)

        t, k_rem = select(merge_hist(), k_rem)
        prefix = lax.shift_left(prefix, 8) | t

        if level < 3:
            def compactl_body(i, off, src=src, dst=dst, shift=shift,
                              m_cand=m_cand, t=t):
                key = src[pl.ds(i * L, L)]
                valid = (i * L + lane) < m_cand
                match = jnp.logical_and(
                    valid, (lax.shift_right_logical(key, shift) & 0xFF) == t)
                plsc.store_compressed(dst.at[pl.ds(off, L)], key, mask=match)
                return off + plsc.all_reduce_population_count(match)[0]
            m_cand = lax.fori_loop(0, nblk, compactl_body, jnp.int32(0))

    bmask = jnp.where(prefix < 0, jnp.int32(MINT32), jnp.int32(-1))
    tbits = jnp.broadcast_to(prefix ^ bmask, (L,))
    thresh = plsc.bitcast(tbits, jnp.float32)

    def mask_body(i, _):
        for u in range(U):
            v = row_v[pl.ds((i * U + u) * L, L)]
            row_v[pl.ds((i * U + u) * L, L)] = jnp.where(
                v >= thresh, v, jnp.float32(-1e9))
        return 0
    lax.fori_loop(0, N // (L * U), mask_body, 0)

    pltpu.sync_copy(row_v, out_hbm.at[wid])


_sc_topk_mask = functools.partial(
    pl.kernel,
    out_type=jax.ShapeDtypeStruct((R, N), jnp.float32),
    mesh=plsc.VectorSubcoreMesh(
        core_axis_name="c", subcore_axis_name="s",
        num_cores=NC, num_subcores=NS),
    compiler_params=pltpu.CompilerParams(needs_layout_passes=False),
    scratch_types=[
        pltpu.VMEM((N,), jnp.float32),
        pltpu.VMEM((HIST_WORDS,), jnp.int32),
        pltpu.VMEM((NB,), jnp.int32),
        pltpu.VMEM((L,), jnp.int32),
        pltpu.VMEM((N + L,), jnp.int32),
        pltpu.VMEM((N + L,), jnp.int32),
    ],
)(_sc_body)


def kernel(scores, k):
    kvec = jnp.full((L,), k, jnp.int32)
    return _sc_topk_mask(scores, kvec)


# speedup vs baseline: 3.9342x; 1.2958x over previous
import functools

import jax
import jax.numpy as jnp
from jax import lax
from jax.experimental import pallas as pl
from jax.experimental.pallas import tpu as pltpu
from jax.experimental.pallas import tpu_sc as plsc

R, N, L = 32, 32768, 16
NB = 256
NBP = NB + 1
NC, NS = 2, 16
MINT32 = -2**31
HIST_WORDS = 4224


def _sortable_key(v):
    b = plsc.bitcast(v, jnp.int32)
    m = (b >> 31) | jnp.int32(MINT32)
    return b ^ m


def _suffix_pick(v, k):
    sfx = lax.rev(plsc.cumsum(lax.rev(v, (0,))), (0,))
    m = sfx >= k
    cnt = plsc.all_reduce_population_count(m)[0]
    idx = cnt - 1
    onehot = lax.iota(jnp.int32, 16) == idx
    val = jnp.sum(jnp.where(onehot, v, 0))
    sfx_i = jnp.sum(jnp.where(onehot, sfx, 0))
    kp = k - (sfx_i - val)
    return idx, kp, val


def _sc_body(scores_hbm, kvec_hbm, out_hbm, row_v, hist_v, total_v, kv_v,
             c1_v, c2_v):
    wid = lax.axis_index("s") * NC + lax.axis_index("c")
    pltpu.sync_copy(scores_hbm.at[wid], row_v)
    pltpu.sync_copy(kvec_hbm, kv_v)
    k_rem = kv_v[...][0]

    lane = lax.iota(jnp.int32, L)
    ones = jnp.ones((L,), jnp.int32)
    zeros16 = jnp.zeros((16,), jnp.int32)
    U = 8

    def zero_hist():
        @plsc.parallel_loop(0, HIST_WORDS // 16, unroll=U)
        def _(i):
            hist_v[pl.ds(i * 16, 16)] = zeros16

    def merge_hist():
        def merge_body(c, chunks):
            acc = zeros16
            for l in range(L):
                acc = acc + hist_v[pl.ds(l * NBP + c * 16, 16)]
            total_v[pl.ds(c * 16, 16)] = acc
            return jnp.where(lane == c, jnp.sum(acc), chunks)
        return lax.fori_loop(0, NB // 16, merge_body, zeros16)

    def select(chunks, k):
        cstar, kp, _ = _suffix_pick(chunks, k)
        v = total_v[pl.ds(cstar * 16, 16)]
        t_loc, knext, _ = _suffix_pick(v, kp)
        return cstar * 16 + t_loc, knext

    zero_hist()

    lane_off = lane * NBP

    @plsc.parallel_loop(0, N // L, unroll=U)
    def _(i):
        key = _sortable_key(row_v[pl.ds(i * L, L)])
        bins = lax.shift_right_logical(key, 24)
        plsc.addupdate_scatter(hist_v, [lane_off + bins], ones)

    t, k_rem = select(merge_hist(), k_rem)
    prefix = t

    def compact0_body(i, off):
        for u in range(U):
            key = _sortable_key(row_v[pl.ds((i * U + u) * L, L)])
            match = lax.shift_right_logical(key, 24) == prefix
            plsc.store_compressed(c1_v.at[pl.ds(off, L)], key, mask=match)
            off = off + plsc.all_reduce_population_count(match)[0]
        return off
    m_cand = lax.fori_loop(0, N // (L * U), compact0_body, jnp.int32(0))

    bufs = (c1_v, c2_v)
    for level in range(1, 4):
        shift = 24 - 8 * level
        src, dst = bufs[(level - 1) % 2], bufs[level % 2]
        nblk = (m_cand + (L - 1)) // L
        zero_hist()

        @plsc.parallel_loop(0, nblk, unroll=2)
        def _(i, src=src, shift=shift, m_cand=m_cand):
            key = src[pl.ds(i * L, L)]
            valid = (i * L + lane) < m_cand
            bins = lax.shift_right_logical(key, shift) & 0xFF
            plsc.addupdate_scatter(hist_v, [lane_off + bins], ones,
                                   mask=valid)

        t, k_rem = select(merge_hist(), k_rem)
        prefix = lax.shift_left(prefix, 8) | t

        if level < 3:
            def compactl_body(i, off, src=src, dst=dst, shift=shift,
                              m_cand=m_cand, t=t):
                key = src[pl.ds(i * L, L)]
                valid = (i * L + lane) < m_cand
                match = jnp.logical_and(
                    valid, (lax.shift_right_logical(key, shift) & 0xFF) == t)
                plsc.store_compressed(dst.at[pl.ds(off, L)], key, mask=match)
                return off + plsc.all_reduce_population_count(match)[0]
            m_cand = lax.fori_loop(0, nblk, compactl_body, jnp.int32(0))

    bmask = jnp.where(prefix < 0, jnp.int32(MINT32), jnp.int32(-1))
    tbits = jnp.broadcast_to(prefix ^ bmask, (L,))
    thresh = plsc.bitcast(tbits, jnp.float32)

    @plsc.parallel_loop(0, N // L, unroll=U)
    def _(i):
        v = row_v[pl.ds(i * L, L)]
        row_v[pl.ds(i * L, L)] = jnp.where(v >= thresh, v, jnp.float32(-1e9))

    pltpu.sync_copy(row_v, out_hbm.at[wid])


_sc_topk_mask = functools.partial(
    pl.kernel,
    out_type=jax.ShapeDtypeStruct((R, N), jnp.float32),
    mesh=plsc.VectorSubcoreMesh(
        core_axis_name="c", subcore_axis_name="s",
        num_cores=NC, num_subcores=NS),
    compiler_params=pltpu.CompilerParams(needs_layout_passes=False),
    scratch_types=[
        pltpu.VMEM((N,), jnp.float32),
        pltpu.VMEM((HIST_WORDS,), jnp.int32),
        pltpu.VMEM((NB,), jnp.int32),
        pltpu.VMEM((L,), jnp.int32),
        pltpu.VMEM((N + L,), jnp.int32),
        pltpu.VMEM((N + L,), jnp.int32),
    ],
)(_sc_body)


def kernel(scores, k):
    kvec = jnp.full((L,), k, jnp.int32)
    return _sc_topk_mask(scores, kvec)


# speedup vs baseline: 6.2146x; 1.5797x over previous
import functools

import jax
import jax.numpy as jnp
from jax import lax
from jax.experimental import pallas as pl
from jax.experimental.pallas import tpu as pltpu
from jax.experimental.pallas import tpu_sc as plsc

R, N, L = 32, 32768, 16
NB = 256
NBP = NB + 1
NC, NS = 2, 16
MINT32 = -2**31
HIST_WORDS = 4224


def _sortable_key(v):
    b = plsc.bitcast(v, jnp.int32)
    m = (b >> 31) | jnp.int32(MINT32)
    return b ^ m


def _suffix_pick(v, k):
    sfx = lax.rev(plsc.cumsum(lax.rev(v, (0,))), (0,))
    m = sfx >= k
    cnt = plsc.all_reduce_population_count(m)[0]
    idx = cnt - 1
    onehot = lax.iota(jnp.int32, 16) == idx
    val = jnp.sum(jnp.where(onehot, v, 0))
    sfx_i = jnp.sum(jnp.where(onehot, sfx, 0))
    kp = k - (sfx_i - val)
    return idx, kp, val


def _sc_body(scores_hbm, kvec_hbm, out_hbm, row_v, hist_v, total_v, kv_v,
             c1_v, c2_v):
    wid = lax.axis_index("s") * NC + lax.axis_index("c")
    pltpu.sync_copy(scores_hbm.at[wid], row_v)
    pltpu.sync_copy(kvec_hbm, kv_v)
    k_rem = kv_v[...][0]

    lane = lax.iota(jnp.int32, L)
    ones = jnp.ones((L,), jnp.int32)
    zeros16 = jnp.zeros((16,), jnp.int32)
    U = 8

    def zero_hist():
        @plsc.parallel_loop(0, HIST_WORDS // 16, unroll=U)
        def _(i):
            hist_v[pl.ds(i * 16, 16)] = zeros16

    def merge_hist():
        def merge_body(c, chunks):
            acc = zeros16
            for l in range(L):
                acc = acc + hist_v[pl.ds(l * NBP + c * 16, 16)]
            total_v[pl.ds(c * 16, 16)] = acc
            return jnp.where(lane == c, jnp.sum(acc), chunks)
        return lax.fori_loop(0, NB // 16, merge_body, zeros16)

    def select(chunks, k):
        cstar, kp, _ = _suffix_pick(chunks, k)
        v = total_v[pl.ds(cstar * 16, 16)]
        t_loc, knext, _ = _suffix_pick(v, kp)
        return cstar * 16 + t_loc, knext

    zero_hist()

    lane_off = lane * NBP

    @plsc.parallel_loop(0, N // L, unroll=U)
    def _(i):
        key = _sortable_key(row_v[pl.ds(i * L, L)])
        bins = lax.shift_right_logical(key, 24)
        plsc.addupdate_scatter(hist_v, [lane_off + bins], ones)

    t, k_rem = select(merge_hist(), k_rem)
    prefix = t

    @plsc.parallel_loop(0, N // L, unroll=U, carry=jnp.int32(0))
    def compact0_loop(i, off):
        key = _sortable_key(row_v[pl.ds(i * L, L)])
        match = lax.shift_right_logical(key, 24) == prefix
        plsc.store_compressed(c1_v.at[pl.ds(off, L)], key, mask=match)
        return off + plsc.all_reduce_population_count(match)[0]
    m_cand = compact0_loop

    bufs = (c1_v, c2_v)
    for level in range(1, 4):
        shift = 24 - 8 * level
        src, dst = bufs[(level - 1) % 2], bufs[level % 2]
        nblk = (m_cand + (L - 1)) // L
        zero_hist()

        @plsc.parallel_loop(0, nblk, unroll=2)
        def _(i, src=src, shift=shift, m_cand=m_cand):
            key = src[pl.ds(i * L, L)]
            valid = (i * L + lane) < m_cand
            bins = lax.shift_right_logical(key, shift) & 0xFF
            plsc.addupdate_scatter(hist_v, [lane_off + bins], ones,
                                   mask=valid)

        t, k_rem = select(merge_hist(), k_rem)
        prefix = lax.shift_left(prefix, 8) | t

        if level < 3:
            @plsc.parallel_loop(0, nblk, unroll=2, carry=jnp.int32(0))
            def compactl_loop(i, off, src=src, dst=dst, shift=shift,
                              m_cand=m_cand, t=t):
                key = src[pl.ds(i * L, L)]
                valid = (i * L + lane) < m_cand
                match = jnp.logical_and(
                    valid, (lax.shift_right_logical(key, shift) & 0xFF) == t)
                plsc.store_compressed(dst.at[pl.ds(off, L)], key, mask=match)
                return off + plsc.all_reduce_population_count(match)[0]
            m_cand = compactl_loop

    bmask = jnp.where(prefix < 0, jnp.int32(MINT32), jnp.int32(-1))
    tbits = jnp.broadcast_to(prefix ^ bmask, (L,))
    thresh = plsc.bitcast(tbits, jnp.float32)

    @plsc.parallel_loop(0, N // L, unroll=U)
    def _(i):
        v = row_v[pl.ds(i * L, L)]
        row_v[pl.ds(i * L, L)] = jnp.where(v >= thresh, v, jnp.float32(-1e9))

    pltpu.sync_copy(row_v, out_hbm.at[wid])


_sc_topk_mask = functools.partial(
    pl.kernel,
    out_type=jax.ShapeDtypeStruct((R, N), jnp.float32),
    mesh=plsc.VectorSubcoreMesh(
        core_axis_name="c", subcore_axis_name="s",
        num_cores=NC, num_subcores=NS),
    compiler_params=pltpu.CompilerParams(needs_layout_passes=False),
    scratch_types=[
        pltpu.VMEM((N,), jnp.float32),
        pltpu.VMEM((HIST_WORDS,), jnp.int32),
        pltpu.VMEM((NB,), jnp.int32),
        pltpu.VMEM((L,), jnp.int32),
        pltpu.VMEM((N + L,), jnp.int32),
        pltpu.VMEM((N + L,), jnp.int32),
    ],
)(_sc_body)


def kernel(scores, k):
    kvec = jnp.full((L,), k, jnp.int32)
    return _sc_topk_mask(scores, kvec)


# speedup vs baseline: 6.2347x; 1.0032x over previous
import functools

import jax
import jax.numpy as jnp
from jax import lax
from jax.experimental import pallas as pl
from jax.experimental.pallas import tpu as pltpu
from jax.experimental.pallas import tpu_sc as plsc

R, N, L = 32, 32768, 16
NB = 256
NBP = NB + 1
NC, NS = 2, 16
MINT32 = -2**31
HIST_WORDS = 4224


def _sortable_key(v):
    b = plsc.bitcast(v, jnp.int32)
    m = (b >> 31) | jnp.int32(MINT32)
    return b ^ m


def _suffix_pick(v, k):
    sfx = lax.rev(plsc.cumsum(lax.rev(v, (0,))), (0,))
    m = sfx >= k
    cnt = plsc.all_reduce_population_count(m)[0]
    idx = cnt - 1
    onehot = lax.iota(jnp.int32, 16) == idx
    val = jnp.sum(jnp.where(onehot, v, 0))
    sfx_i = jnp.sum(jnp.where(onehot, sfx, 0))
    kp = k - (sfx_i - val)
    return idx, kp, val


NCHUNK = 8
CW = N // NCHUNK


def _sc_body(scores_hbm, kvec_hbm, out_hbm, row_v, hist_v, total_v, kv_v,
             c1_v, c2_v, *sems):
    wid = lax.axis_index("s") * NC + lax.axis_index("c")
    in_copies = [
        pltpu.async_copy(scores_hbm.at[wid, pl.ds(c * CW, CW)],
                         row_v.at[pl.ds(c * CW, CW)], sems[c])
        for c in range(NCHUNK)
    ]
    pltpu.sync_copy(kvec_hbm, kv_v)
    k_rem = kv_v[...][0]

    lane = lax.iota(jnp.int32, L)
    ones = jnp.ones((L,), jnp.int32)
    zeros16 = jnp.zeros((16,), jnp.int32)
    U = 8

    def zero_hist():
        @plsc.parallel_loop(0, HIST_WORDS // 16, unroll=U)
        def _(i):
            hist_v[pl.ds(i * 16, 16)] = zeros16

    def merge_hist():
        def merge_body(c, chunks):
            vs = [hist_v[pl.ds(l * NBP + c * 16, 16)] for l in range(L)]
            while len(vs) > 1:
                vs = [a + b for a, b in zip(vs[::2], vs[1::2])]
            total_v[pl.ds(c * 16, 16)] = vs[0]
            return jnp.where(lane == c, jnp.sum(vs[0]), chunks)
        return lax.fori_loop(0, NB // 16, merge_body, zeros16)

    def select(chunks, k):
        cstar, kp, _ = _suffix_pick(chunks, k)
        v = total_v[pl.ds(cstar * 16, 16)]
        t_loc, knext, _ = _suffix_pick(v, kp)
        return cstar * 16 + t_loc, knext

    zero_hist()

    lane_off = lane * NBP

    for c in range(NCHUNK):
        in_copies[c].wait()

        @plsc.parallel_loop(c * (CW // L), (c + 1) * (CW // L), unroll=U)
        def _(i):
            key = _sortable_key(row_v[pl.ds(i * L, L)])
            bins = lax.shift_right_logical(key, 24)
            plsc.addupdate_scatter(hist_v, [lane_off + bins], ones)

    t, k_rem = select(merge_hist(), k_rem)
    prefix = t

    @plsc.parallel_loop(0, N // L, unroll=U, carry=jnp.int32(0))
    def compact0_loop(i, off):
        key = _sortable_key(row_v[pl.ds(i * L, L)])
        match = lax.shift_right_logical(key, 24) == prefix
        plsc.store_compressed(c1_v.at[pl.ds(off, L)], key, mask=match)
        return off + plsc.all_reduce_population_count(match)[0]
    m_cand = compact0_loop

    bufs = (c1_v, c2_v)
    for level in range(1, 4):
        shift = 24 - 8 * level
        src, dst = bufs[(level - 1) % 2], bufs[level % 2]
        nblk = (m_cand + (L - 1)) // L
        zero_hist()

        @plsc.parallel_loop(0, nblk, unroll=2)
        def _(i, src=src, shift=shift, m_cand=m_cand):
            key = src[pl.ds(i * L, L)]
            valid = (i * L + lane) < m_cand
            bins = lax.shift_right_logical(key, shift) & 0xFF
            plsc.addupdate_scatter(hist_v, [lane_off + bins], ones,
                                   mask=valid)

        t, k_rem = select(merge_hist(), k_rem)
        prefix = lax.shift_left(prefix, 8) | t

        if level < 3:
            @plsc.parallel_loop(0, nblk, unroll=2, carry=jnp.int32(0))
            def compactl_loop(i, off, src=src, dst=dst, shift=shift,
                              m_cand=m_cand, t=t):
                key = src[pl.ds(i * L, L)]
                valid = (i * L + lane) < m_cand
                match = jnp.logical_and(
                    valid, (lax.shift_right_logical(key, shift) & 0xFF) == t)
                plsc.store_compressed(dst.at[pl.ds(off, L)], key, mask=match)
                return off + plsc.all_reduce_population_count(match)[0]
            m_cand = compactl_loop

    bmask = jnp.where(prefix < 0, jnp.int32(MINT32), jnp.int32(-1))
    tbits = jnp.broadcast_to(prefix ^ bmask, (L,))
    thresh = plsc.bitcast(tbits, jnp.float32)

    out_copies = []
    for c in range(NCHUNK):
        @plsc.parallel_loop(c * (CW // L), (c + 1) * (CW // L), unroll=U)
        def _(i):
            v = row_v[pl.ds(i * L, L)]
            row_v[pl.ds(i * L, L)] = jnp.where(
                v >= thresh, v, jnp.float32(-1e9))

        out_copies.append(
            pltpu.async_copy(row_v.at[pl.ds(c * CW, CW)],
                             out_hbm.at[wid, pl.ds(c * CW, CW)], sems[c]))

    for h in out_copies:
        h.wait()


_sc_topk_mask = functools.partial(
    pl.kernel,
    out_type=jax.ShapeDtypeStruct((R, N), jnp.float32),
    mesh=plsc.VectorSubcoreMesh(
        core_axis_name="c", subcore_axis_name="s",
        num_cores=NC, num_subcores=NS),
    compiler_params=pltpu.CompilerParams(needs_layout_passes=False),
    scratch_types=[
        pltpu.VMEM((N,), jnp.float32),
        pltpu.VMEM((HIST_WORDS,), jnp.int32),
        pltpu.VMEM((NB,), jnp.int32),
        pltpu.VMEM((L,), jnp.int32),
        pltpu.VMEM((N + L,), jnp.int32),
        pltpu.VMEM((N + L,), jnp.int32),
    ] + [pltpu.SemaphoreType.DMA] * NCHUNK,
)(_sc_body)


def kernel(scores, k):
    kvec = jnp.full((L,), k, jnp.int32)
    return _sc_topk_mask(scores, kvec)
